# baseline (speedup 1.0000x reference)
import jax
import jax.numpy as jnp
from jax.experimental import pallas as pl
from jax.experimental.pallas import tpu as pltpu

_V, _L, _D, _C, _B = 50000, 512, 300, 512, 64
_NCH = 3
_DC = _NCH * 128
_KB = 7680
_NK = (_C * _D) // _KB
_VB = 2048
_NPB = 25


def _pack_kernel(et_ref, eye_ref, out_ref):
    r = jax.lax.dot_general(et_ref[...], eye_ref[...], (((0,), (0,)), ((), ())),
                            preferred_element_type=jnp.float32)
    w0 = pltpu.pack_elementwise([r[:, 0:128], r[:, 128:256]],
                                packed_dtype=jnp.bfloat16)
    w1 = pltpu.pack_elementwise([r[:, 256:384], jnp.zeros_like(r[:, 0:128])],
                                packed_dtype=jnp.bfloat16)
    out_ref[pl.Slice(0, _VB, 2), :] = w0
    out_ref[pl.Slice(1, _VB, 2), :] = w1


def _attn_kernel(idx_ref, tab_ref, pec_ref, w2_ref, out_ref, h2_ref):
    j, k = pl.program_id(0), pl.program_id(1)
    half = _B // 2

    def gather(bb, hb):
        base = bb * _L
        for t in range(_L):
            row = pl.multiple_of(idx_ref[base + t], 2)
            slab = pltpu.bitcast(tab_ref[pl.ds(row, 2), :], jnp.bfloat16)
            hb[3 * t:3 * t + 3, :] = slab[:_NCH, :].astype(jnp.float32)

    @pl.when(k == 0)
    def _prologue():
        gather(j * half, h2_ref.at[0])

    sel = jax.lax.rem(k, 2)
    b_next = j * half + jnp.minimum(k + 1, half - 1)
    gather(b_next, h2_ref.at[1 - sel])

    hb = h2_ref.at[sel]
    h = jnp.concatenate(
        [hb[pl.Slice(j2, _L, _NCH), :] + pec_ref[j2 * _L:(j2 + 1) * _L, :]
         for j2 in range(_NCH)], axis=1)
    simt = jax.lax.dot_general(w2_ref[...], h, (((1,), (1,)), ((), ())),
                               preferred_element_type=jnp.float32)
    e = jnp.exp(simt)
    s = jnp.sum(e, axis=1, keepdims=True)
    p = e * (1.0 / s)
    ctx = jax.lax.dot_general(p, h, (((1,), (0,)), ((), ())),
                              preferred_element_type=jnp.float32)
    out_ref[0] = ctx[:, :_D]


def _fc_kernel(x_ref, w_ref, b_ref, out_ref, acc_ref):
    k = pl.program_id(1)

    @pl.when(k == 0)
    def _init():
        acc_ref[...] = jnp.zeros_like(acc_ref)

    acc_ref[...] += jax.lax.dot_general(
        x_ref[...], w_ref[...], (((1,), (1,)), ((), ())),
        preferred_element_type=jnp.float32)

    @pl.when(k == _NK - 1)
    def _fin():
        out_ref[...] = acc_ref[...] + b_ref[...]


def kernel(x, embed_w, coef, pe, conv_w, conv_b, fc_w, fc_b):
    coef = coef.astype(jnp.float32)
    idx2 = (x.reshape(-1) * 2).astype(jnp.int32)

    eye = (1.0 - coef) * jnp.eye(_D, _DC, dtype=jnp.float32)
    tab = pl.pallas_call(
        _pack_kernel,
        grid=(2, (_NPB + 1) // 2),
        in_specs=[
            pl.BlockSpec((_D, _VB),
                         lambda j, k: (0, jnp.minimum(j * 13 + k, _NPB - 1))),
            pl.BlockSpec(memory_space=pltpu.VMEM),
        ],
        out_specs=pl.BlockSpec(
            (2 * _VB, 128), lambda j, k: (jnp.minimum(j * 13 + k, _NPB - 1), 0)),
        out_shape=jax.ShapeDtypeStruct((2 * _V, 128), jnp.int32),
        compiler_params=pltpu.CompilerParams(
            dimension_semantics=("parallel", "arbitrary"),
            vmem_limit_bytes=56 * 1024 * 1024,
        ),
    )(embed_w.T, eye)

    pe_aug = jnp.concatenate(
        [coef * pe, jnp.ones((_L, 1), jnp.float32),
         jnp.zeros((_L, _DC - _D - 1), jnp.float32)], axis=1)
    pec = pe_aug.reshape(_L, _NCH, 128).transpose(1, 0, 2).reshape(_NCH * _L, 128)

    w2 = jnp.concatenate(
        [conv_w, conv_b[:, None],
         jnp.zeros((_C, _DC - _D - 1), jnp.float32)], axis=1)

    grid_spec = pltpu.PrefetchScalarGridSpec(
        num_scalar_prefetch=1,
        grid=(2, _B // 2),
        in_specs=[
            pl.BlockSpec(memory_space=pltpu.VMEM),
            pl.BlockSpec(memory_space=pltpu.VMEM),
            pl.BlockSpec(memory_space=pltpu.VMEM),
        ],
        out_specs=pl.BlockSpec((1, _C, _D),
                               lambda j, k, *_: (j * (_B // 2) + k, 0, 0)),
        scratch_shapes=[pltpu.VMEM((2, _NCH * _L, 128), jnp.float32)],
    )
    ctx = pl.pallas_call(
        _attn_kernel,
        grid_spec=grid_spec,
        out_shape=jax.ShapeDtypeStruct((_B, _C, _D), jnp.float32),
        compiler_params=pltpu.CompilerParams(
            dimension_semantics=("parallel", "arbitrary"),
            vmem_limit_bytes=56 * 1024 * 1024,
        ),
    )(idx2, tab, pec, w2)

    out = pl.pallas_call(
        _fc_kernel,
        grid=(2, _NK),
        in_specs=[
            pl.BlockSpec((_B, _KB), lambda j, k: (0, k)),
            pl.BlockSpec((_C // 2, _KB), lambda j, k: (j, k)),
            pl.BlockSpec((1, _C // 2), lambda j, k: (0, j)),
        ],
        out_specs=pl.BlockSpec((_B, _C // 2), lambda j, k: (0, j)),
        out_shape=jax.ShapeDtypeStruct((_B, _C), jnp.float32),
        scratch_shapes=[pltpu.VMEM((_B, _C // 2), jnp.float32)],
        compiler_params=pltpu.CompilerParams(
            dimension_semantics=("parallel", "arbitrary"),
            vmem_limit_bytes=56 * 1024 * 1024,
        ),
    )(ctx.reshape(_B, _C * _D), fc_w, fc_b.reshape(1, _C))
    return out


# speedup vs baseline: 1.0213x; 1.0213x over previous
import jax
import jax.numpy as jnp
from jax.experimental import pallas as pl
from jax.experimental.pallas import tpu as pltpu

_V, _L, _D, _C, _B = 50000, 512, 300, 512, 64
_NCH = 3
_DC = _NCH * 128
_KB = 15360
_NK = (_C * _D) // _KB
_VB = 4096
_NPB = 13


def _pack_kernel(et_ref, eye_ref, out_ref):
    r = jax.lax.dot_general(et_ref[...], eye_ref[...], (((0,), (0,)), ((), ())),
                            preferred_element_type=jnp.float32)
    w0 = pltpu.pack_elementwise([r[:, 0:128], r[:, 128:256]],
                                packed_dtype=jnp.bfloat16)
    w1 = pltpu.pack_elementwise([r[:, 256:384], jnp.zeros_like(r[:, 0:128])],
                                packed_dtype=jnp.bfloat16)
    out_ref[pl.Slice(0, _VB, 2), :] = w0
    out_ref[pl.Slice(1, _VB, 2), :] = w1


def _attn_kernel(idx_ref, tab_ref, pec_ref, w2_ref, out_ref, h2_ref):
    j, k = pl.program_id(0), pl.program_id(1)
    half = _B // 2

    def gather(bb, hb):
        base = bb * _L
        for t in range(_L):
            row = pl.multiple_of(idx_ref[base + t], 2)
            slab = pltpu.bitcast(tab_ref[pl.ds(row, 2), :], jnp.bfloat16)
            hb[3 * t:3 * t + 3, :] = slab[:_NCH, :].astype(jnp.float32)

    gather(j * half + k, h2_ref.at[0])

    hb = h2_ref.at[0]
    h = jnp.concatenate(
        [hb[pl.Slice(j2, _L, _NCH), :] + pec_ref[j2 * _L:(j2 + 1) * _L, :]
         for j2 in range(_NCH)], axis=1)
    simt = jax.lax.dot_general(w2_ref[...], h, (((1,), (1,)), ((), ())),
                               preferred_element_type=jnp.float32)
    e = jnp.exp(simt)
    s = jnp.sum(e, axis=1, keepdims=True)
    p = e * (1.0 / s)
    ctx = jax.lax.dot_general(p, h, (((1,), (0,)), ((), ())),
                              preferred_element_type=jnp.float32)
    out_ref[0] = ctx[:, :_D]


def _fc_kernel(x_ref, w_ref, b_ref, out_ref, acc_ref):
    k = pl.program_id(1)

    @pl.when(k == 0)
    def _init():
        acc_ref[...] = jnp.zeros_like(acc_ref)

    acc_ref[...] += jax.lax.dot_general(
        x_ref[...], w_ref[...], (((1,), (1,)), ((), ())),
        preferred_element_type=jnp.float32)

    @pl.when(k == _NK - 1)
    def _fin():
        out_ref[...] = acc_ref[...] + b_ref[...]


def kernel(x, embed_w, coef, pe, conv_w, conv_b, fc_w, fc_b):
    coef = coef.astype(jnp.float32)
    idx2 = (x.reshape(-1) * 2).astype(jnp.int32)

    eye = (1.0 - coef) * jnp.eye(_D, _DC, dtype=jnp.float32)
    tab = pl.pallas_call(
        _pack_kernel,
        grid=(2, (_NPB + 1) // 2),
        in_specs=[
            pl.BlockSpec((_D, _VB),
                         lambda j, k: (0, jnp.minimum(j * ((_NPB + 1) // 2) + k, _NPB - 1))),
            pl.BlockSpec(memory_space=pltpu.VMEM),
        ],
        out_specs=pl.BlockSpec(
            (2 * _VB, 128), lambda j, k: (jnp.minimum(j * ((_NPB + 1) // 2) + k, _NPB - 1), 0)),
        out_shape=jax.ShapeDtypeStruct((2 * _V, 128), jnp.int32),
        compiler_params=pltpu.CompilerParams(
            dimension_semantics=("parallel", "arbitrary"),
            vmem_limit_bytes=56 * 1024 * 1024,
        ),
    )(embed_w.T, eye)

    pe_aug = jnp.concatenate(
        [coef * pe, jnp.ones((_L, 1), jnp.float32),
         jnp.zeros((_L, _DC - _D - 1), jnp.float32)], axis=1)
    pec = pe_aug.reshape(_L, _NCH, 128).transpose(1, 0, 2).reshape(_NCH * _L, 128)

    w2 = jnp.concatenate(
        [conv_w, conv_b[:, None],
         jnp.zeros((_C, _DC - _D - 1), jnp.float32)], axis=1)

    grid_spec = pltpu.PrefetchScalarGridSpec(
        num_scalar_prefetch=1,
        grid=(2, _B // 2),
        in_specs=[
            pl.BlockSpec(memory_space=pltpu.VMEM),
            pl.BlockSpec(memory_space=pltpu.VMEM),
            pl.BlockSpec(memory_space=pltpu.VMEM),
        ],
        out_specs=pl.BlockSpec((1, _C, _D),
                               lambda j, k, *_: (j * (_B // 2) + k, 0, 0)),
        scratch_shapes=[pltpu.VMEM((1, _NCH * _L, 128), jnp.float32)],
    )
    ctx = pl.pallas_call(
        _attn_kernel,
        grid_spec=grid_spec,
        out_shape=jax.ShapeDtypeStruct((_B, _C, _D), jnp.float32),
        compiler_params=pltpu.CompilerParams(
            dimension_semantics=("parallel", "arbitrary"),
            vmem_limit_bytes=56 * 1024 * 1024,
        ),
    )(idx2, tab, pec, w2)

    out = pl.pallas_call(
        _fc_kernel,
        grid=(2, _NK),
        in_specs=[
            pl.BlockSpec((_B, _KB), lambda j, k: (0, k)),
            pl.BlockSpec((_C // 2, _KB), lambda j, k: (j, k)),
            pl.BlockSpec((1, _C // 2), lambda j, k: (0, j)),
        ],
        out_specs=pl.BlockSpec((_B, _C // 2), lambda j, k: (0, j)),
        out_shape=jax.ShapeDtypeStruct((_B, _C), jnp.float32),
        scratch_shapes=[pltpu.VMEM((_B, _C // 2), jnp.float32)],
        compiler_params=pltpu.CompilerParams(
            dimension_semantics=("parallel", "arbitrary"),
            vmem_limit_bytes=56 * 1024 * 1024,
        ),
    )(ctx.reshape(_B, _C * _D), fc_w, fc_b.reshape(1, _C))
    return out


# speedup vs baseline: 1.0220x; 1.0008x over previous
import jax
import jax.numpy as jnp
from jax.experimental import pallas as pl
from jax.experimental.pallas import tpu as pltpu

_V, _L, _D, _C, _B = 50000, 512, 300, 512, 64
_NCH = 3
_DC = _NCH * 128
_KB = 15360
_NK = (_C * _D) // _KB
_VB = 4096
_NPB = 13


def _pack_kernel(et_ref, eye_ref, out_ref):
    r = jax.lax.dot_general(et_ref[...], eye_ref[...], (((0,), (0,)), ((), ())),
                            preferred_element_type=jnp.float32)
    w0 = pltpu.pack_elementwise([r[:, 0:128], r[:, 128:256]],
                                packed_dtype=jnp.bfloat16)
    w1 = pltpu.pack_elementwise([r[:, 256:384], jnp.zeros_like(r[:, 0:128])],
                                packed_dtype=jnp.bfloat16)
    out_ref[pl.Slice(0, _VB, 2), :] = w0
    out_ref[pl.Slice(1, _VB, 2), :] = w1


def _attn_kernel(idx_ref, tab_ref, pec_ref, w2_ref, out_ref, h2_ref):
    def gather(bb, hb):
        base = bb * _L
        for t in range(_L):
            row = pl.multiple_of(idx_ref[base + t], 2)
            slab = pltpu.bitcast(tab_ref[pl.ds(row, 2), :], jnp.bfloat16)
            hb[3 * t:3 * t + 3, :] = slab[:_NCH, :].astype(jnp.float32)

    gather(pl.program_id(0), h2_ref.at[0])

    hb = h2_ref.at[0]
    h = jnp.concatenate(
        [hb[pl.Slice(j2, _L, _NCH), :] + pec_ref[j2 * _L:(j2 + 1) * _L, :]
         for j2 in range(_NCH)], axis=1)
    simt = jax.lax.dot_general(w2_ref[...], h, (((1,), (1,)), ((), ())),
                               preferred_element_type=jnp.float32)
    e = jnp.exp(simt)
    s = jnp.sum(e, axis=1, keepdims=True)
    p = e * (1.0 / s)
    ctx = jax.lax.dot_general(p, h, (((1,), (0,)), ((), ())),
                              preferred_element_type=jnp.float32)
    out_ref[0] = ctx[:, :_D]


def _fc_kernel(x_ref, w_ref, b_ref, out_ref, acc_ref):
    k = pl.program_id(1)

    @pl.when(k == 0)
    def _init():
        acc_ref[...] = jnp.zeros_like(acc_ref)

    acc_ref[...] += jax.lax.dot_general(
        x_ref[...], w_ref[...], (((1,), (1,)), ((), ())),
        preferred_element_type=jnp.float32)

    @pl.when(k == _NK - 1)
    def _fin():
        out_ref[...] = acc_ref[...] + b_ref[...]


def kernel(x, embed_w, coef, pe, conv_w, conv_b, fc_w, fc_b):
    coef = coef.astype(jnp.float32)
    idx2 = (x.reshape(-1) * 2).astype(jnp.int32)

    eye = (1.0 - coef) * jnp.eye(_D, _DC, dtype=jnp.float32)
    tab = pl.pallas_call(
        _pack_kernel,
        grid=(2, (_NPB + 1) // 2),
        in_specs=[
            pl.BlockSpec((_D, _VB),
                         lambda j, k: (0, jnp.minimum(j * ((_NPB + 1) // 2) + k, _NPB - 1))),
            pl.BlockSpec(memory_space=pltpu.VMEM),
        ],
        out_specs=pl.BlockSpec(
            (2 * _VB, 128), lambda j, k: (jnp.minimum(j * ((_NPB + 1) // 2) + k, _NPB - 1), 0)),
        out_shape=jax.ShapeDtypeStruct((2 * _V, 128), jnp.int32),
        compiler_params=pltpu.CompilerParams(
            dimension_semantics=("parallel", "arbitrary"),
            vmem_limit_bytes=56 * 1024 * 1024,
        ),
    )(embed_w.T, eye)

    pe_aug = jnp.concatenate(
        [coef * pe, jnp.ones((_L, 1), jnp.float32),
         jnp.zeros((_L, _DC - _D - 1), jnp.float32)], axis=1)
    pec = pe_aug.reshape(_L, _NCH, 128).transpose(1, 0, 2).reshape(_NCH * _L, 128)

    w2 = jnp.concatenate(
        [conv_w, conv_b[:, None],
         jnp.zeros((_C, _DC - _D - 1), jnp.float32)], axis=1)

    grid_spec = pltpu.PrefetchScalarGridSpec(
        num_scalar_prefetch=1,
        grid=(_B,),
        in_specs=[
            pl.BlockSpec(memory_space=pltpu.VMEM),
            pl.BlockSpec(memory_space=pltpu.VMEM),
            pl.BlockSpec(memory_space=pltpu.VMEM),
        ],
        out_specs=pl.BlockSpec((1, _C, _D), lambda b, *_: (b, 0, 0)),
        scratch_shapes=[pltpu.VMEM((1, _NCH * _L, 128), jnp.float32)],
    )
    ctx = pl.pallas_call(
        _attn_kernel,
        grid_spec=grid_spec,
        out_shape=jax.ShapeDtypeStruct((_B, _C, _D), jnp.float32),
        compiler_params=pltpu.CompilerParams(
            dimension_semantics=("parallel",),
            vmem_limit_bytes=56 * 1024 * 1024,
            flags={"XLA_TPU_STORE_TO_LOAD_FORWARDING_WINDOW": 12288},
        ),
    )(idx2, tab, pec, w2)

    out = pl.pallas_call(
        _fc_kernel,
        grid=(2, _NK),
        in_specs=[
            pl.BlockSpec((_B, _KB), lambda j, k: (0, k)),
            pl.BlockSpec((_C // 2, _KB), lambda j, k: (j, k)),
            pl.BlockSpec((1, _C // 2), lambda j, k: (0, j)),
        ],
        out_specs=pl.BlockSpec((_B, _C // 2), lambda j, k: (0, j)),
        out_shape=jax.ShapeDtypeStruct((_B, _C), jnp.float32),
        scratch_shapes=[pltpu.VMEM((_B, _C // 2), jnp.float32)],
        compiler_params=pltpu.CompilerParams(
            dimension_semantics=("parallel", "arbitrary"),
            vmem_limit_bytes=56 * 1024 * 1024,
        ),
    )(ctx.reshape(_B, _C * _D), fc_w, fc_b.reshape(1, _C))
    return out


# speedup vs baseline: 1.0392x; 1.0168x over previous
import jax
import jax.numpy as jnp
from jax.experimental import pallas as pl
from jax.experimental.pallas import tpu as pltpu

_V, _L, _D, _C, _B = 50000, 512, 300, 512, 64
_NCH = 3
_DC = _NCH * 128
_KB = 15360
_NK = (_C * _D) // _KB
_VB = 4096
_NPB = 13


def _pack_kernel(et_ref, eye_ref, out_ref):
    r = jax.lax.dot_general(et_ref[...], eye_ref[...], (((0,), (0,)), ((), ())),
                            preferred_element_type=jnp.float32)
    w0 = pltpu.pack_elementwise([r[:, 0:128], r[:, 128:256]],
                                packed_dtype=jnp.bfloat16)
    w1 = pltpu.pack_elementwise([r[:, 256:384], jnp.zeros_like(r[:, 0:128])],
                                packed_dtype=jnp.bfloat16)
    out_ref[pl.Slice(0, _VB, 2), :] = w0
    out_ref[pl.Slice(1, _VB, 2), :] = w1


def _attn_kernel(idx_ref, tab_ref, pec_ref, w2_ref, out_ref, h2_ref):
    def gather(bb, hb):
        base = bb * _L
        for t in range(_L):
            row = pl.multiple_of(idx_ref[base + t], 2)
            slab = pltpu.bitcast(tab_ref[pl.ds(row, 2), :], jnp.bfloat16)
            hb[3 * t:3 * t + 3, :] = slab[:_NCH, :].astype(jnp.float32)

    gather(pl.program_id(0), h2_ref.at[0])

    hb = h2_ref.at[0]
    h = jnp.concatenate(
        [hb[pl.Slice(j2, _L, _NCH), :] + pec_ref[j2 * _L:(j2 + 1) * _L, :]
         for j2 in range(_NCH)], axis=1)
    simt = jax.lax.dot_general(w2_ref[...], h, (((1,), (1,)), ((), ())),
                               preferred_element_type=jnp.float32)
    e = jnp.exp(simt)
    s = jnp.sum(e, axis=1, keepdims=True)
    ctxu = jax.lax.dot_general(e, h, (((1,), (0,)), ((), ())),
                               preferred_element_type=jnp.float32)
    ctx = ctxu * (1.0 / s)
    out_ref[0] = ctx[:, :_D]


def _fc_kernel(x_ref, w_ref, b_ref, out_ref, acc_ref):
    k = pl.program_id(1)

    @pl.when(k == 0)
    def _init():
        acc_ref[...] = jnp.zeros_like(acc_ref)

    acc_ref[...] += jax.lax.dot_general(
        x_ref[...], w_ref[...], (((1,), (1,)), ((), ())),
        preferred_element_type=jnp.float32)

    @pl.when(k == _NK - 1)
    def _fin():
        out_ref[...] = acc_ref[...] + b_ref[...]


def kernel(x, embed_w, coef, pe, conv_w, conv_b, fc_w, fc_b):
    coef = coef.astype(jnp.float32)
    idx2 = (x.reshape(-1) * 2).astype(jnp.int32)

    eye = (1.0 - coef) * jnp.eye(_D, _DC, dtype=jnp.float32)
    tab = pl.pallas_call(
        _pack_kernel,
        grid=(2, (_NPB + 1) // 2),
        in_specs=[
            pl.BlockSpec((_D, _VB),
                         lambda j, k: (0, jnp.minimum(j * ((_NPB + 1) // 2) + k, _NPB - 1))),
            pl.BlockSpec(memory_space=pltpu.VMEM),
        ],
        out_specs=pl.BlockSpec(
            (2 * _VB, 128), lambda j, k: (jnp.minimum(j * ((_NPB + 1) // 2) + k, _NPB - 1), 0)),
        out_shape=jax.ShapeDtypeStruct((2 * _V, 128), jnp.int32),
        compiler_params=pltpu.CompilerParams(
            dimension_semantics=("parallel", "arbitrary"),
            vmem_limit_bytes=56 * 1024 * 1024,
        ),
    )(embed_w.T, eye)

    pe_aug = jnp.concatenate(
        [coef * pe, jnp.ones((_L, 1), jnp.float32),
         jnp.zeros((_L, _DC - _D - 1), jnp.float32)], axis=1)
    pec = pe_aug.reshape(_L, _NCH, 128).transpose(1, 0, 2).reshape(_NCH * _L, 128)

    w2 = jnp.concatenate(
        [conv_w, conv_b[:, None],
         jnp.zeros((_C, _DC - _D - 1), jnp.float32)], axis=1)

    grid_spec = pltpu.PrefetchScalarGridSpec(
        num_scalar_prefetch=1,
        grid=(_B,),
        in_specs=[
            pl.BlockSpec(memory_space=pltpu.VMEM),
            pl.BlockSpec(memory_space=pltpu.VMEM),
            pl.BlockSpec(memory_space=pltpu.VMEM),
        ],
        out_specs=pl.BlockSpec((1, _C, _D), lambda b, *_: (b, 0, 0)),
        scratch_shapes=[pltpu.VMEM((1, _NCH * _L, 128), jnp.float32)],
    )
    ctx = pl.pallas_call(
        _attn_kernel,
        grid_spec=grid_spec,
        out_shape=jax.ShapeDtypeStruct((_B, _C, _D), jnp.float32),
        compiler_params=pltpu.CompilerParams(
            dimension_semantics=("parallel",),
            vmem_limit_bytes=56 * 1024 * 1024,
            flags={"XLA_TPU_STORE_TO_LOAD_FORWARDING_WINDOW": 12288},
        ),
    )(idx2, tab, pec, w2)

    out = pl.pallas_call(
        _fc_kernel,
        grid=(2, _NK),
        in_specs=[
            pl.BlockSpec((_B, _KB), lambda j, k: (0, k)),
            pl.BlockSpec((_C // 2, _KB), lambda j, k: (j, k)),
            pl.BlockSpec((1, _C // 2), lambda j, k: (0, j)),
        ],
        out_specs=pl.BlockSpec((_B, _C // 2), lambda j, k: (0, j)),
        out_shape=jax.ShapeDtypeStruct((_B, _C), jnp.float32),
        scratch_shapes=[pltpu.VMEM((_B, _C // 2), jnp.float32)],
        compiler_params=pltpu.CompilerParams(
            dimension_semantics=("parallel", "arbitrary"),
            vmem_limit_bytes=56 * 1024 * 1024,
        ),
    )(ctx.reshape(_B, _C * _D), fc_w, fc_b.reshape(1, _C))
    return out


# speedup vs baseline: 1.0746x; 1.0341x over previous
import jax
import jax.numpy as jnp
from jax.experimental import pallas as pl
from jax.experimental.pallas import tpu as pltpu

_V, _L, _D, _C, _B = 50000, 512, 300, 512, 64
_NCH = 3
_DC = _NCH * 128
_KB = 7680
_NK = (_C * _D) // _KB
_VB = 4096
_NPB = 13


def _pack_kernel(et_ref, eye_ref, out_ref):
    r = jax.lax.dot_general(et_ref[...], eye_ref[...], (((0,), (0,)), ((), ())),
                            preferred_element_type=jnp.float32)
    w0 = pltpu.pack_elementwise([r[:, 0:128], r[:, 128:256]],
                                packed_dtype=jnp.bfloat16)
    w1 = pltpu.pack_elementwise([r[:, 256:384], jnp.zeros_like(r[:, 0:128])],
                                packed_dtype=jnp.bfloat16)
    out_ref[pl.Slice(0, _VB, 2), :] = w0
    out_ref[pl.Slice(1, _VB, 2), :] = w1


def _attn_kernel(idx_ref, tab_ref, pec_ref, w2_ref, out_ref, h2_ref):
    def gather(bb, hb):
        base = bb * _L
        for t in range(_L):
            row = pl.multiple_of(idx_ref[base + t], 2)
            slab = pltpu.bitcast(tab_ref[pl.ds(row, 2), :], jnp.bfloat16)
            hb[3 * t:3 * t + 3, :] = slab[:_NCH, :].astype(jnp.float32)

    gather(pl.program_id(0), h2_ref.at[0])

    hb = h2_ref.at[0]
    h = jnp.concatenate(
        [hb[pl.Slice(j2, _L, _NCH), :] + pec_ref[j2 * _L:(j2 + 1) * _L, :]
         for j2 in range(_NCH)], axis=1)
    simt = jax.lax.dot_general(w2_ref[...], h, (((1,), (1,)), ((), ())),
                               preferred_element_type=jnp.float32)
    e = jnp.exp(simt)
    s = jnp.sum(e, axis=1, keepdims=True)
    ctxu = jax.lax.dot_general(e, h, (((1,), (0,)), ((), ())),
                               preferred_element_type=jnp.float32)
    ctx = ctxu * (1.0 / s)
    out_ref[0] = ctx[:, :_D]


def _fc_kernel(x_ref, w_ref, b_ref, out_ref, acc_ref):
    k = pl.program_id(0)

    @pl.when(k == 0)
    def _init():
        acc_ref[...] = jnp.zeros_like(acc_ref)

    acc_ref[...] += jax.lax.dot_general(
        x_ref[...], w_ref[...], (((1,), (1,)), ((), ())),
        preferred_element_type=jnp.float32)

    @pl.when(k == _NK - 1)
    def _fin():
        out_ref[...] = acc_ref[...] + b_ref[...]


def kernel(x, embed_w, coef, pe, conv_w, conv_b, fc_w, fc_b):
    coef = coef.astype(jnp.float32)
    idx2 = (x.reshape(-1) * 2).astype(jnp.int32)

    eye = (1.0 - coef) * jnp.eye(_D, _DC, dtype=jnp.float32)
    tab = pl.pallas_call(
        _pack_kernel,
        grid=(2, (_NPB + 1) // 2),
        in_specs=[
            pl.BlockSpec((_D, _VB),
                         lambda j, k: (0, jnp.minimum(j * ((_NPB + 1) // 2) + k, _NPB - 1))),
            pl.BlockSpec(memory_space=pltpu.VMEM),
        ],
        out_specs=pl.BlockSpec(
            (2 * _VB, 128), lambda j, k: (jnp.minimum(j * ((_NPB + 1) // 2) + k, _NPB - 1), 0)),
        out_shape=jax.ShapeDtypeStruct((2 * _V, 128), jnp.int32),
        compiler_params=pltpu.CompilerParams(
            dimension_semantics=("parallel", "arbitrary"),
            vmem_limit_bytes=56 * 1024 * 1024,
        ),
    )(embed_w.T, eye)

    pe_aug = jnp.concatenate(
        [coef * pe, jnp.ones((_L, 1), jnp.float32),
         jnp.zeros((_L, _DC - _D - 1), jnp.float32)], axis=1)
    pec = pe_aug.reshape(_L, _NCH, 128).transpose(1, 0, 2).reshape(_NCH * _L, 128)

    w2 = jnp.concatenate(
        [conv_w, conv_b[:, None],
         jnp.zeros((_C, _DC - _D - 1), jnp.float32)], axis=1)

    grid_spec = pltpu.PrefetchScalarGridSpec(
        num_scalar_prefetch=1,
        grid=(_B,),
        in_specs=[
            pl.BlockSpec(memory_space=pltpu.VMEM),
            pl.BlockSpec(memory_space=pltpu.VMEM),
            pl.BlockSpec(memory_space=pltpu.VMEM),
        ],
        out_specs=pl.BlockSpec((1, _C, _D), lambda b, *_: (b, 0, 0)),
        scratch_shapes=[pltpu.VMEM((1, _NCH * _L, 128), jnp.float32)],
    )
    ctx = pl.pallas_call(
        _attn_kernel,
        grid_spec=grid_spec,
        out_shape=jax.ShapeDtypeStruct((_B, _C, _D), jnp.float32),
        compiler_params=pltpu.CompilerParams(
            dimension_semantics=("parallel",),
            vmem_limit_bytes=56 * 1024 * 1024,
            flags={"XLA_TPU_STORE_TO_LOAD_FORWARDING_WINDOW": 12288},
        ),
    )(idx2, tab, pec, w2)

    out = pl.pallas_call(
        _fc_kernel,
        grid=(_NK,),
        in_specs=[
            pl.BlockSpec((_B, _KB), lambda k: (0, k)),
            pl.BlockSpec((_C, _KB), lambda k: (0, k)),
            pl.BlockSpec((1, _C), lambda k: (0, 0)),
        ],
        out_specs=pl.BlockSpec((_B, _C), lambda k: (0, 0)),
        out_shape=jax.ShapeDtypeStruct((_B, _C), jnp.float32),
        scratch_shapes=[pltpu.VMEM((_B, _C), jnp.float32)],
        compiler_params=pltpu.CompilerParams(
            dimension_semantics=("arbitrary",),
            vmem_limit_bytes=56 * 1024 * 1024,
        ),
    )(ctx.reshape(_B, _C * _D), fc_w, fc_b.reshape(1, _C))
    return out


# speedup vs baseline: 1.0753x; 1.0006x over previous
import jax
import jax.numpy as jnp
from jax.experimental import pallas as pl
from jax.experimental.pallas import tpu as pltpu

_V, _L, _D, _C, _B = 50000, 512, 300, 512, 64
_NCH = 3
_DC = _NCH * 128
_KB = 7680
_NK = (_C * _D) // _KB
_VB = 4096
_NPB = 13


def _pack_kernel(et_ref, eye_ref, out_ref):
    r = jax.lax.dot_general(et_ref[...], eye_ref[...], (((0,), (0,)), ((), ())),
                            preferred_element_type=jnp.float32)
    w0 = pltpu.pack_elementwise([r[:, 0:128], r[:, 128:256]],
                                packed_dtype=jnp.bfloat16)
    w1 = pltpu.pack_elementwise([r[:, 256:384], jnp.zeros_like(r[:, 0:128])],
                                packed_dtype=jnp.bfloat16)
    out_ref[pl.Slice(0, _VB, 2), :] = w0
    out_ref[pl.Slice(1, _VB, 2), :] = w1


def _attn_kernel(idx_ref, tab_ref, pec_ref, w2_ref, out_ref, h2_ref):
    def gather(bb, hb):
        base = bb * _L
        for t in range(_L):
            row = pl.multiple_of(idx_ref[base + t], 2)
            slab = pltpu.bitcast(tab_ref[pl.ds(row, 2), :], jnp.bfloat16)
            hb[3 * t:3 * t + 3, :] = slab[:_NCH, :].astype(jnp.float32)

    gather(pl.program_id(0), h2_ref.at[0])

    hb = h2_ref.at[0]
    h = jnp.concatenate(
        [hb[pl.Slice(j2, _L, _NCH), :] + pec_ref[j2 * _L:(j2 + 1) * _L, :]
         for j2 in range(_NCH)], axis=1)
    simt = jax.lax.dot_general(w2_ref[...], h, (((1,), (1,)), ((), ())),
                               preferred_element_type=jnp.float32)
    e = jnp.exp(simt)
    s = jnp.sum(e, axis=1, keepdims=True)
    ctxu = jax.lax.dot_general(e, h, (((1,), (0,)), ((), ())),
                               preferred_element_type=jnp.float32)
    ctx = ctxu * (1.0 / s)
    out_ref[0] = ctx[:, :_D]


def _fc_kernel(x_ref, w_ref, b_ref, out_ref, acc_ref):
    k = pl.program_id(0)

    @pl.when(k == 0)
    def _init():
        acc_ref[...] = jnp.zeros_like(acc_ref)

    acc_ref[...] += jax.lax.dot_general(
        x_ref[...], w_ref[...], (((1,), (1,)), ((), ())),
        preferred_element_type=jnp.float32)

    @pl.when(k == _NK - 1)
    def _fin():
        out_ref[...] = acc_ref[...] + b_ref[...]


def kernel(x, embed_w, coef, pe, conv_w, conv_b, fc_w, fc_b):
    coef = coef.astype(jnp.float32)
    idx2 = (x.reshape(-1) * 2).astype(jnp.int32)

    eye = (1.0 - coef) * jnp.eye(_D, _DC, dtype=jnp.float32)
    tab = pl.pallas_call(
        _pack_kernel,
        grid=(2, (_NPB + 1) // 2),
        in_specs=[
            pl.BlockSpec((_D, _VB),
                         lambda j, k: (0, jnp.minimum(j * ((_NPB + 1) // 2) + k, _NPB - 1))),
            pl.BlockSpec(memory_space=pltpu.VMEM),
        ],
        out_specs=pl.BlockSpec(
            (2 * _VB, 128), lambda j, k: (jnp.minimum(j * ((_NPB + 1) // 2) + k, _NPB - 1), 0)),
        out_shape=jax.ShapeDtypeStruct((2 * _V, 128), jnp.int32),
        compiler_params=pltpu.CompilerParams(
            dimension_semantics=("parallel", "arbitrary"),
            vmem_limit_bytes=56 * 1024 * 1024,
        ),
    )(embed_w.T, eye)

    pe_aug = jnp.concatenate(
        [coef * pe, jnp.ones((_L, 1), jnp.float32),
         jnp.zeros((_L, _DC - _D - 1), jnp.float32)], axis=1)
    pec = pe_aug.reshape(_L, _NCH, 128).transpose(1, 0, 2).reshape(_NCH * _L, 128)

    w2 = jnp.concatenate(
        [conv_w, conv_b[:, None],
         jnp.zeros((_C, _DC - _D - 1), jnp.float32)], axis=1)

    grid_spec = pltpu.PrefetchScalarGridSpec(
        num_scalar_prefetch=1,
        grid=(_B,),
        in_specs=[
            pl.BlockSpec(memory_space=pltpu.VMEM),
            pl.BlockSpec(memory_space=pltpu.VMEM),
            pl.BlockSpec(memory_space=pltpu.VMEM),
        ],
        out_specs=pl.BlockSpec((1, _C, _D), lambda b, *_: (b, 0, 0)),
        scratch_shapes=[pltpu.VMEM((1, _NCH * _L, 128), jnp.float32)],
    )
    ctx = pl.pallas_call(
        _attn_kernel,
        grid_spec=grid_spec,
        out_shape=jax.ShapeDtypeStruct((_B, _C, _D), jnp.float32),
        compiler_params=pltpu.CompilerParams(
            dimension_semantics=("parallel",),
            vmem_limit_bytes=56 * 1024 * 1024,
        ),
    )(idx2, tab, pec, w2)

    out = pl.pallas_call(
        _fc_kernel,
        grid=(_NK,),
        in_specs=[
            pl.BlockSpec((_B, _KB), lambda k: (0, k)),
            pl.BlockSpec((_C, _KB), lambda k: (0, k)),
            pl.BlockSpec((1, _C), lambda k: (0, 0)),
        ],
        out_specs=pl.BlockSpec((_B, _C), lambda k: (0, 0)),
        out_shape=jax.ShapeDtypeStruct((_B, _C), jnp.float32),
        scratch_shapes=[pltpu.VMEM((_B, _C), jnp.float32)],
        compiler_params=pltpu.CompilerParams(
            dimension_semantics=("arbitrary",),
            vmem_limit_bytes=56 * 1024 * 1024,
        ),
    )(ctx.reshape(_B, _C * _D), fc_w, fc_b.reshape(1, _C))
    return out
